# baseline (speedup 1.0000x reference)
import jax
import jax.numpy as jnp
from jax.experimental import pallas as pl
from jax.experimental.pallas import tpu as pltpu

_H = 64


def _fused_gru_kernel(scal_ref, wt_ref, bt_ref, x_ref, h_ref, d_ref, wx_ref,
                      wh_ref, b_ref, out_ref, hnew_ref):
    i = pl.program_id(0)
    nsteps = pl.num_programs(0)

    @pl.when(i == 0)
    def _init():
        out_ref[...] = jnp.full(out_ref.shape, -jnp.inf, jnp.float32)

    h = h_ref[...]
    g = (jax.lax.dot_general(
            x_ref[...].astype(jnp.bfloat16), wx_ref[...],
            (((1,), (0,)), ((), ())), preferred_element_type=jnp.float32)
         + jax.lax.dot_general(
            h.astype(jnp.bfloat16), wh_ref[...],
            (((1,), (0,)), ((), ())), preferred_element_type=jnp.float32)
         + b_ref[...])
    r = jax.nn.sigmoid(g[:, 0:128])
    z = jax.nn.sigmoid(g[:, 128:256])
    n = jnp.tanh(g[:, 256:384] + r * g[:, 384:512])
    h_all = (1.0 - z) * n + z * h

    d = d_ref[...]
    dmax_a = jnp.max(d[:, 0:3], axis=1, keepdims=True)
    dmax_b = jnp.max(d[:, 3:6], axis=1, keepdims=True)
    lane = jax.lax.broadcasted_iota(jnp.int32, h_all.shape, 1)
    dmax = jnp.where(lane < _H, dmax_a, dmax_b)
    mask = (dmax > 0.0) & (scal_ref[1] > 0.0)
    hnew_ref[...] = jnp.where(mask, h_all, 0.0)

    block_max = jnp.max(jnp.where(mask, h_all, -jnp.inf), axis=0,
                        keepdims=True)
    out_ref[...] = jnp.maximum(out_ref[...], block_max)

    @pl.when(i == nsteps - 1)
    def _finalize():
        inv = 1.0 / jnp.log(scal_ref[0] + jnp.exp(1.0))
        tf = jnp.tanh(inv * wt_ref[...] + bt_ref[...])
        m = jnp.maximum(out_ref[:, 0:_H], out_ref[:, _H:2 * _H])
        mm = jnp.concatenate([m, m], axis=1)
        out_ref[...] = mm + tf


def kernel(interval, t, co_embeddings, divided, no_embeddings,
           unrelated_embeddings, is_last, hidden_state, W_ih, W_hh, b_ih,
           b_hh, W_t, b_t):
    N, G = co_embeddings.shape
    H = W_hh.shape[1]
    if hidden_state is None:
        hidden_state = jnp.zeros((N, H), co_embeddings.dtype)
    Np = N // 2

    WiT = W_ih.T
    WhT = W_hh.T
    ZH = jnp.zeros((H, H), jnp.float32)
    Wr, Wz, Wn = WiT[:, :H], WiT[:, H:2 * H], WiT[:, 2 * H:]
    Vr, Vz, Vn = WhT[:, :H], WhT[:, H:2 * H], WhT[:, 2 * H:]
    Wx2 = jnp.concatenate([
        jnp.concatenate([Wr, ZH, Wz, ZH, Wn, ZH, ZH, ZH], axis=1),
        jnp.concatenate([ZH, Wr, ZH, Wz, ZH, Wn, ZH, ZH], axis=1),
    ], axis=0).astype(jnp.bfloat16)
    Wh2 = jnp.concatenate([
        jnp.concatenate([Vr, ZH, Vz, ZH, ZH, ZH, Vn, ZH], axis=1),
        jnp.concatenate([ZH, Vr, ZH, Vz, ZH, ZH, ZH, Vn], axis=1),
    ], axis=0).astype(jnp.bfloat16)
    br = b_ih[:H] + b_hh[:H]
    bz = b_ih[H:2 * H] + b_hh[H:2 * H]
    bn_i = b_ih[2 * H:]
    bn_h = b_hh[2 * H:]
    bbig = jnp.concatenate([br, br, bz, bz, bn_i, bn_i, bn_h, bn_h])[None, :]

    scalars = jnp.stack([
        jnp.asarray(interval, jnp.float32),
        jnp.logical_not(is_last).astype(jnp.float32)])
    wt_row = jnp.concatenate([W_t.T, W_t.T], axis=1).astype(jnp.float32)
    bt_row = jnp.concatenate([b_t, b_t])[None, :]

    x2 = co_embeddings.reshape(Np, 2 * G)
    h2 = hidden_state.reshape(Np, 2 * H)
    d2 = divided.reshape(Np, 6)

    B = 2000
    grid = Np // B

    out_small, h_new2 = pl.pallas_call(
        _fused_gru_kernel,
        grid=(grid,),
        in_specs=[
            pl.BlockSpec(memory_space=pltpu.SMEM),
            pl.BlockSpec((1, 2 * H), lambda i: (0, 0)),
            pl.BlockSpec((1, 2 * H), lambda i: (0, 0)),
            pl.BlockSpec((B, 2 * G), lambda i: (i, 0)),
            pl.BlockSpec((B, 2 * H), lambda i: (i, 0)),
            pl.BlockSpec((B, 6), lambda i: (i, 0)),
            pl.BlockSpec((2 * G, 8 * H), lambda i: (0, 0)),
            pl.BlockSpec((2 * H, 8 * H), lambda i: (0, 0)),
            pl.BlockSpec((1, 8 * H), lambda i: (0, 0)),
        ],
        out_specs=[
            pl.BlockSpec((1, 2 * H), lambda i: (0, 0)),
            pl.BlockSpec((B, 2 * H), lambda i: (i, 0)),
        ],
        out_shape=[
            jax.ShapeDtypeStruct((1, 2 * H), jnp.float32),
            jax.ShapeDtypeStruct((Np, 2 * H), jnp.float32),
        ],
        compiler_params=pltpu.CompilerParams(
            dimension_semantics=("arbitrary",)),
    )(scalars, wt_row, bt_row, x2, h2, d2, Wx2, Wh2, bbig)

    return (out_small[0, :H], h_new2.reshape(N, H))


# speedup vs baseline: 5.4505x; 5.4505x over previous
import jax
import jax.numpy as jnp
from jax.experimental import pallas as pl
from jax.experimental.pallas import tpu as pltpu

_H = 64


def _fused_gru_kernel(scal_ref, wt_ref, bt_ref, x_ref, h_ref, d_ref, a_ref,
                      b_ref, out_ref, hnew_ref):
    i = pl.program_id(0)
    nsteps = pl.num_programs(0)
    n_total = jnp.int32(scal_ref[2])

    @pl.when(i == 0)
    def _init():
        out_ref[...] = jnp.full(out_ref.shape, -jnp.inf, jnp.float32)

    h = h_ref[...]
    xh = jnp.concatenate([x_ref[...].astype(jnp.bfloat16),
                          h.astype(jnp.bfloat16)], axis=0)
    g = jax.lax.dot_general(
        a_ref[...], xh, (((1,), (0,)), ((), ())),
        preferred_element_type=jnp.float32) + b_ref[...]
    r = jax.nn.sigmoid(g[0:_H])
    z = jax.nn.sigmoid(g[_H:2 * _H])
    n = jnp.tanh(g[2 * _H:3 * _H] + r * g[3 * _H:4 * _H])
    h_all = (1.0 - z) * n + z * h

    dmax = jnp.max(d_ref[...], axis=0, keepdims=True)
    col = i * x_ref.shape[1] + jax.lax.broadcasted_iota(
        jnp.int32, dmax.shape, 1)
    mask = (dmax > 0.0) & (scal_ref[1] > 0.0) & (col < n_total)
    hnew_ref[...] = jnp.where(mask, h_all, 0.0)

    block_max = jnp.max(jnp.where(mask, h_all, -jnp.inf), axis=1,
                        keepdims=True)
    out_ref[...] = jnp.maximum(out_ref[...], block_max)

    @pl.when(i == nsteps - 1)
    def _finalize():
        inv = 1.0 / jnp.log(scal_ref[0] + jnp.exp(1.0))
        out_ref[...] = out_ref[...] + jnp.tanh(inv * wt_ref[...] + bt_ref[...])


def kernel(interval, t, co_embeddings, divided, no_embeddings,
           unrelated_embeddings, is_last, hidden_state, W_ih, W_hh, b_ih,
           b_hh, W_t, b_t):
    N, G = co_embeddings.shape
    H = W_hh.shape[1]
    if hidden_state is None:
        hidden_state = jnp.zeros((N, H), co_embeddings.dtype)

    WiT = W_ih.T
    WhT = W_hh.T
    ZG = jnp.zeros((G, H), jnp.float32)
    ZH = jnp.zeros((H, H), jnp.float32)
    A = jnp.concatenate([
        jnp.concatenate([WiT[:, :H], WiT[:, H:2 * H], WiT[:, 2 * H:], ZG],
                        axis=1),
        jnp.concatenate([WhT[:, :H], WhT[:, H:2 * H], ZH, WhT[:, 2 * H:]],
                        axis=1),
    ], axis=0)
    A_T = A.T.astype(jnp.bfloat16)
    bcol = jnp.concatenate([
        b_ih[:H] + b_hh[:H], b_ih[H:2 * H] + b_hh[H:2 * H],
        b_ih[2 * H:], b_hh[2 * H:]])[:, None]

    scalars = jnp.stack([
        jnp.asarray(interval, jnp.float32),
        jnp.logical_not(is_last).astype(jnp.float32),
        jnp.float32(N)])
    wt_col = W_t.astype(jnp.float32)
    bt_col = b_t[:, None]

    xT = co_embeddings.T
    hT = hidden_state.T
    dT = divided.T

    B = 4096
    grid = -(-N // B)

    out_col, h_newT = pl.pallas_call(
        _fused_gru_kernel,
        grid=(grid,),
        in_specs=[
            pl.BlockSpec(memory_space=pltpu.SMEM),
            pl.BlockSpec((H, 1), lambda i: (0, 0)),
            pl.BlockSpec((H, 1), lambda i: (0, 0)),
            pl.BlockSpec((G, B), lambda i: (0, i)),
            pl.BlockSpec((H, B), lambda i: (0, i)),
            pl.BlockSpec((3, B), lambda i: (0, i)),
            pl.BlockSpec((4 * H, G + H), lambda i: (0, 0)),
            pl.BlockSpec((4 * H, 1), lambda i: (0, 0)),
        ],
        out_specs=[
            pl.BlockSpec((H, 1), lambda i: (0, 0)),
            pl.BlockSpec((H, B), lambda i: (0, i)),
        ],
        out_shape=[
            jax.ShapeDtypeStruct((H, 1), jnp.float32),
            jax.ShapeDtypeStruct((H, N), jnp.float32),
        ],
        compiler_params=pltpu.CompilerParams(
            dimension_semantics=("arbitrary",)),
    )(scalars, wt_col, bt_col, xT, hT, dT, A_T, bcol)

    return (out_col[:, 0], h_newT.T)


# speedup vs baseline: 6.1065x; 1.1204x over previous
import jax
import jax.numpy as jnp
from jax.experimental import pallas as pl
from jax.experimental.pallas import tpu as pltpu

_H = 64


def _fused_gru_kernel(scal_ref, wt_ref, bt_ref, x_ref, h_ref, d_ref, a_ref,
                      b_ref, out_ref, hnew_ref):
    i = pl.program_id(0)
    nsteps = pl.num_programs(0)
    n_total = jnp.int32(scal_ref[2])

    @pl.when(i == 0)
    def _init():
        out_ref[...] = jnp.full(out_ref.shape, -jnp.inf, jnp.float32)

    h = h_ref[...]
    xh = jnp.concatenate([x_ref[...].astype(jnp.bfloat16),
                          h.astype(jnp.bfloat16)], axis=0)
    g = jax.lax.dot_general(
        a_ref[...], xh, (((1,), (0,)), ((), ())),
        preferred_element_type=jnp.float32) + b_ref[...]
    r = jax.nn.sigmoid(g[0:_H])
    z = jax.nn.sigmoid(g[_H:2 * _H])
    n = jnp.tanh(g[2 * _H:3 * _H] + r * g[3 * _H:4 * _H])
    h_all = n + z * (h - n)

    dmax = jnp.max(d_ref[...], axis=0, keepdims=True)
    col = i * x_ref.shape[1] + jax.lax.broadcasted_iota(
        jnp.int32, dmax.shape, 1)
    mask = (dmax > 0.0) & (scal_ref[1] > 0.0) & (col < n_total)
    hnew_ref[...] = jnp.where(mask, h_all, 0.0)

    block_max = jnp.max(jnp.where(mask, h_all, -jnp.inf), axis=1,
                        keepdims=True)
    out_ref[...] = jnp.maximum(out_ref[...], block_max)

    @pl.when(i == nsteps - 1)
    def _finalize():
        inv = 1.0 / jnp.log(scal_ref[0] + jnp.exp(1.0))
        out_ref[...] = out_ref[...] + jnp.tanh(inv * wt_ref[...] + bt_ref[...])


def kernel(interval, t, co_embeddings, divided, no_embeddings,
           unrelated_embeddings, is_last, hidden_state, W_ih, W_hh, b_ih,
           b_hh, W_t, b_t):
    N, G = co_embeddings.shape
    H = W_hh.shape[1]
    if hidden_state is None:
        hidden_state = jnp.zeros((N, H), co_embeddings.dtype)

    WiT = W_ih.T
    WhT = W_hh.T
    ZG = jnp.zeros((G, H), jnp.float32)
    ZH = jnp.zeros((H, H), jnp.float32)
    A = jnp.concatenate([
        jnp.concatenate([WiT[:, :H], WiT[:, H:2 * H], WiT[:, 2 * H:], ZG],
                        axis=1),
        jnp.concatenate([WhT[:, :H], WhT[:, H:2 * H], ZH, WhT[:, 2 * H:]],
                        axis=1),
    ], axis=0)
    A_T = A.T.astype(jnp.bfloat16)
    bcol = jnp.concatenate([
        b_ih[:H] + b_hh[:H], b_ih[H:2 * H] + b_hh[H:2 * H],
        b_ih[2 * H:], b_hh[2 * H:]])[:, None]

    scalars = jnp.stack([
        jnp.asarray(interval, jnp.float32),
        jnp.logical_not(is_last).astype(jnp.float32),
        jnp.float32(N)])
    wt_col = W_t.astype(jnp.float32)
    bt_col = b_t[:, None]

    xT = co_embeddings.T
    hT = hidden_state.T
    dT = divided.T

    B = 6400
    grid = -(-N // B)

    out_col, h_newT = pl.pallas_call(
        _fused_gru_kernel,
        grid=(grid,),
        in_specs=[
            pl.BlockSpec(memory_space=pltpu.SMEM),
            pl.BlockSpec((H, 1), lambda i: (0, 0)),
            pl.BlockSpec((H, 1), lambda i: (0, 0)),
            pl.BlockSpec((G, B), lambda i: (0, i)),
            pl.BlockSpec((H, B), lambda i: (0, i)),
            pl.BlockSpec((3, B), lambda i: (0, i)),
            pl.BlockSpec((4 * H, G + H), lambda i: (0, 0)),
            pl.BlockSpec((4 * H, 1), lambda i: (0, 0)),
        ],
        out_specs=[
            pl.BlockSpec((H, 1), lambda i: (0, 0)),
            pl.BlockSpec((H, B), lambda i: (0, i)),
        ],
        out_shape=[
            jax.ShapeDtypeStruct((H, 1), jnp.float32),
            jax.ShapeDtypeStruct((H, N), jnp.float32),
        ],
        compiler_params=pltpu.CompilerParams(
            dimension_semantics=("arbitrary",)),
    )(scalars, wt_col, bt_col, xT, hT, dT, A_T, bcol)

    return (out_col[:, 0], h_newT.T)


# speedup vs baseline: 6.1541x; 1.0078x over previous
import jax
import jax.numpy as jnp
from jax.experimental import pallas as pl
from jax.experimental.pallas import tpu as pltpu

_H = 64


def _fused_gru_kernel(scal_ref, wt_ref, bt_ref, x_ref, h_ref, d_ref, a_ref,
                      b_ref, out_ref, hnew_ref):
    i = pl.program_id(0)
    nsteps = pl.num_programs(0)
    n_total = jnp.int32(scal_ref[2])

    @pl.when(i == 0)
    def _init():
        out_ref[...] = jnp.full(out_ref.shape, -jnp.inf, jnp.float32)

    h = h_ref[...]
    xh = jnp.concatenate([x_ref[...].astype(jnp.bfloat16),
                          h.astype(jnp.bfloat16)], axis=0)
    g = jax.lax.dot_general(
        a_ref[...], xh, (((1,), (0,)), ((), ())),
        preferred_element_type=jnp.float32) + b_ref[...]
    rz = 0.5 * jnp.tanh(0.5 * g[0:2 * _H]) + 0.5
    r = rz[0:_H]
    z = rz[_H:2 * _H]
    n = jnp.tanh(g[2 * _H:3 * _H] + r * g[3 * _H:4 * _H])
    h_all = n + z * (h - n)

    dmax = jnp.max(d_ref[...], axis=0, keepdims=True)
    col = i * x_ref.shape[1] + jax.lax.broadcasted_iota(
        jnp.int32, dmax.shape, 1)
    mask = (dmax > 0.0) & (scal_ref[1] > 0.0) & (col < n_total)
    hnew_ref[...] = jnp.where(mask, h_all, 0.0)

    block_max = jnp.max(jnp.where(mask, h_all, -jnp.inf), axis=1,
                        keepdims=True)
    out_ref[...] = jnp.maximum(out_ref[...], block_max)

    @pl.when(i == nsteps - 1)
    def _finalize():
        inv = 1.0 / jnp.log(scal_ref[0] + jnp.exp(1.0))
        out_ref[...] = out_ref[...] + jnp.tanh(inv * wt_ref[...] + bt_ref[...])


def kernel(interval, t, co_embeddings, divided, no_embeddings,
           unrelated_embeddings, is_last, hidden_state, W_ih, W_hh, b_ih,
           b_hh, W_t, b_t):
    N, G = co_embeddings.shape
    H = W_hh.shape[1]
    if hidden_state is None:
        hidden_state = jnp.zeros((N, H), co_embeddings.dtype)

    WiT = W_ih.T
    WhT = W_hh.T
    ZG = jnp.zeros((G, H), jnp.float32)
    ZH = jnp.zeros((H, H), jnp.float32)
    A = jnp.concatenate([
        jnp.concatenate([WiT[:, :H], WiT[:, H:2 * H], WiT[:, 2 * H:], ZG],
                        axis=1),
        jnp.concatenate([WhT[:, :H], WhT[:, H:2 * H], ZH, WhT[:, 2 * H:]],
                        axis=1),
    ], axis=0)
    A_T = A.T.astype(jnp.bfloat16)
    bcol = jnp.concatenate([
        b_ih[:H] + b_hh[:H], b_ih[H:2 * H] + b_hh[H:2 * H],
        b_ih[2 * H:], b_hh[2 * H:]])[:, None]

    scalars = jnp.stack([
        jnp.asarray(interval, jnp.float32),
        jnp.logical_not(is_last).astype(jnp.float32),
        jnp.float32(N)])
    wt_col = W_t.astype(jnp.float32)
    bt_col = b_t[:, None]

    xT = co_embeddings.T
    hT = hidden_state.T
    dT = divided.T

    B = 6400
    grid = -(-N // B)

    out_col, h_newT = pl.pallas_call(
        _fused_gru_kernel,
        grid=(grid,),
        in_specs=[
            pl.BlockSpec(memory_space=pltpu.SMEM),
            pl.BlockSpec((H, 1), lambda i: (0, 0)),
            pl.BlockSpec((H, 1), lambda i: (0, 0)),
            pl.BlockSpec((G, B), lambda i: (0, i)),
            pl.BlockSpec((H, B), lambda i: (0, i)),
            pl.BlockSpec((3, B), lambda i: (0, i)),
            pl.BlockSpec((4 * H, G + H), lambda i: (0, 0)),
            pl.BlockSpec((4 * H, 1), lambda i: (0, 0)),
        ],
        out_specs=[
            pl.BlockSpec((H, 1), lambda i: (0, 0)),
            pl.BlockSpec((H, B), lambda i: (0, i)),
        ],
        out_shape=[
            jax.ShapeDtypeStruct((H, 1), jnp.float32),
            jax.ShapeDtypeStruct((H, N), jnp.float32),
        ],
        compiler_params=pltpu.CompilerParams(
            dimension_semantics=("arbitrary",)),
    )(scalars, wt_col, bt_col, xT, hT, dT, A_T, bcol)

    return (out_col[:, 0], h_newT.T)


# speedup vs baseline: 6.6265x; 1.0768x over previous
import jax
import jax.numpy as jnp
from jax.experimental import pallas as pl
from jax.experimental.pallas import tpu as pltpu

_H = 64


def _fused_gru_kernel(scal_ref, wt_ref, bt_ref, x_ref, h_ref, d_ref, a_ref,
                      b_ref, out_ref, hnew_ref):
    i = pl.program_id(0)
    nsteps = pl.num_programs(0)
    n_total = jnp.int32(scal_ref[2])

    @pl.when(i == 0)
    def _init():
        out_ref[...] = jnp.full(out_ref.shape, -jnp.inf, jnp.float32)

    h = h_ref[...]
    xh = jnp.concatenate([x_ref[...].astype(jnp.bfloat16),
                          h.astype(jnp.bfloat16)], axis=0)
    g = jax.lax.dot_general(
        a_ref[...], xh, (((1,), (0,)), ((), ())),
        preferred_element_type=jnp.float32) + b_ref[...]
    rz = 0.5 * jnp.tanh(0.5 * g[0:2 * _H]) + 0.5
    r = rz[0:_H]
    z = rz[_H:2 * _H]
    n = jnp.tanh(g[2 * _H:3 * _H] + r * g[3 * _H:4 * _H])
    h_all = n + z * (h - n)

    dmax = jnp.max(d_ref[...], axis=0, keepdims=True)
    col = i * x_ref.shape[1] + jax.lax.broadcasted_iota(
        jnp.int32, dmax.shape, 1)
    mask = (dmax > 0.0) & (scal_ref[1] > 0.0) & (col < n_total)
    hnew_ref[...] = jnp.where(mask, h_all, 0.0)

    block_max = jnp.max(jnp.where(mask, h_all, -jnp.inf), axis=1,
                        keepdims=True)
    out_ref[...] = jnp.maximum(out_ref[...], block_max)

    @pl.when(i == nsteps - 1)
    def _finalize():
        inv = 1.0 / jnp.log(scal_ref[0] + jnp.exp(1.0))
        out_ref[...] = out_ref[...] + jnp.tanh(inv * wt_ref[...] + bt_ref[...])


def kernel(interval, t, co_embeddings, divided, no_embeddings,
           unrelated_embeddings, is_last, hidden_state, W_ih, W_hh, b_ih,
           b_hh, W_t, b_t):
    N, G = co_embeddings.shape
    H = W_hh.shape[1]
    if hidden_state is None:
        hidden_state = jnp.zeros((N, H), co_embeddings.dtype)

    WiT = W_ih.T
    WhT = W_hh.T
    ZG = jnp.zeros((G, H), jnp.float32)
    ZH = jnp.zeros((H, H), jnp.float32)
    A = jnp.concatenate([
        jnp.concatenate([WiT[:, :H], WiT[:, H:2 * H], WiT[:, 2 * H:], ZG],
                        axis=1),
        jnp.concatenate([WhT[:, :H], WhT[:, H:2 * H], ZH, WhT[:, 2 * H:]],
                        axis=1),
    ], axis=0)
    A_T = A.T.astype(jnp.bfloat16)
    bcol = jnp.concatenate([
        b_ih[:H] + b_hh[:H], b_ih[H:2 * H] + b_hh[H:2 * H],
        b_ih[2 * H:], b_hh[2 * H:]])[:, None]

    scalars = jnp.stack([
        jnp.asarray(interval, jnp.float32),
        jnp.logical_not(is_last).astype(jnp.float32),
        jnp.float32(N)])
    wt_col = W_t.astype(jnp.float32)
    bt_col = b_t[:, None]

    xT = co_embeddings.T
    hT = hidden_state.T
    dT = divided.T

    B = 12800
    grid = -(-N // B)

    out_col, h_newT = pl.pallas_call(
        _fused_gru_kernel,
        grid=(grid,),
        in_specs=[
            pl.BlockSpec(memory_space=pltpu.SMEM),
            pl.BlockSpec((H, 1), lambda i: (0, 0)),
            pl.BlockSpec((H, 1), lambda i: (0, 0)),
            pl.BlockSpec((G, B), lambda i: (0, i)),
            pl.BlockSpec((H, B), lambda i: (0, i)),
            pl.BlockSpec((3, B), lambda i: (0, i)),
            pl.BlockSpec((4 * H, G + H), lambda i: (0, 0)),
            pl.BlockSpec((4 * H, 1), lambda i: (0, 0)),
        ],
        out_specs=[
            pl.BlockSpec((H, 1), lambda i: (0, 0)),
            pl.BlockSpec((H, B), lambda i: (0, i)),
        ],
        out_shape=[
            jax.ShapeDtypeStruct((H, 1), jnp.float32),
            jax.ShapeDtypeStruct((H, N), jnp.float32),
        ],
        compiler_params=pltpu.CompilerParams(
            dimension_semantics=("arbitrary",)),
    )(scalars, wt_col, bt_col, xT, hT, dT, A_T, bcol)

    return (out_col[:, 0], h_newT.T)


# speedup vs baseline: 6.6715x; 1.0068x over previous
import jax
import jax.numpy as jnp
from jax.experimental import pallas as pl
from jax.experimental.pallas import tpu as pltpu

_H = 64


def _fused_gru_kernel(scal_ref, wt_ref, bt_ref, x_ref, h_ref, d_ref, a_ref,
                      b_ref, out_ref, hnew_ref):
    i = pl.program_id(0)
    nsteps = pl.num_programs(0)
    n_total = jnp.int32(scal_ref[2])

    @pl.when(i == 0)
    def _init():
        out_ref[...] = jnp.full(out_ref.shape, -jnp.inf, jnp.float32)

    h = h_ref[...]
    xh = jnp.concatenate([x_ref[...].astype(jnp.bfloat16),
                          h.astype(jnp.bfloat16)], axis=0)
    g = jax.lax.dot_general(
        a_ref[...], xh, (((1,), (0,)), ((), ())),
        preferred_element_type=jnp.float32) + b_ref[...]
    rz = 0.5 * jnp.tanh(0.5 * g[0:2 * _H]) + 0.5
    r = rz[0:_H]
    z = rz[_H:2 * _H]
    n = jnp.tanh(g[2 * _H:3 * _H] + r * g[3 * _H:4 * _H])
    h_all = n + z * (h - n)

    dmax = jnp.max(d_ref[...], axis=0, keepdims=True)
    col = i * x_ref.shape[1] + jax.lax.broadcasted_iota(
        jnp.int32, dmax.shape, 1)
    mask = (dmax > 0.0) & (scal_ref[1] > 0.0) & (col < n_total)
    hnew_ref[...] = jnp.where(mask, h_all, 0.0)

    block_max = jnp.max(jnp.where(mask, h_all, -jnp.inf), axis=1,
                        keepdims=True)
    out_ref[...] = jnp.maximum(out_ref[...], block_max)

    @pl.when(i == nsteps - 1)
    def _finalize():
        inv = 1.0 / jnp.log(scal_ref[0] + jnp.exp(1.0))
        out_ref[...] = out_ref[...] + jnp.tanh(inv * wt_ref[...] + bt_ref[...])


def kernel(interval, t, co_embeddings, divided, no_embeddings,
           unrelated_embeddings, is_last, hidden_state, W_ih, W_hh, b_ih,
           b_hh, W_t, b_t):
    N, G = co_embeddings.shape
    H = W_hh.shape[1]
    if hidden_state is None:
        hidden_state = jnp.zeros((N, H), co_embeddings.dtype)

    WiT = W_ih.T
    WhT = W_hh.T
    ZG = jnp.zeros((G, H), jnp.float32)
    ZH = jnp.zeros((H, H), jnp.float32)
    A = jnp.concatenate([
        jnp.concatenate([WiT[:, :H], WiT[:, H:2 * H], WiT[:, 2 * H:], ZG],
                        axis=1),
        jnp.concatenate([WhT[:, :H], WhT[:, H:2 * H], ZH, WhT[:, 2 * H:]],
                        axis=1),
    ], axis=0)
    A_T = A.T.astype(jnp.bfloat16)
    bcol = jnp.concatenate([
        b_ih[:H] + b_hh[:H], b_ih[H:2 * H] + b_hh[H:2 * H],
        b_ih[2 * H:], b_hh[2 * H:]])[:, None]

    scalars = jnp.stack([
        jnp.asarray(interval, jnp.float32),
        jnp.logical_not(is_last).astype(jnp.float32),
        jnp.float32(N)])
    wt_col = W_t.astype(jnp.float32)
    bt_col = b_t[:, None]

    xT = co_embeddings.T
    hT = hidden_state.T
    dT = divided.T

    B = 14336
    grid = -(-N // B)

    out_col, h_newT = pl.pallas_call(
        _fused_gru_kernel,
        grid=(grid,),
        in_specs=[
            pl.BlockSpec(memory_space=pltpu.SMEM),
            pl.BlockSpec((H, 1), lambda i: (0, 0)),
            pl.BlockSpec((H, 1), lambda i: (0, 0)),
            pl.BlockSpec((G, B), lambda i: (0, i)),
            pl.BlockSpec((H, B), lambda i: (0, i)),
            pl.BlockSpec((3, B), lambda i: (0, i)),
            pl.BlockSpec((4 * H, G + H), lambda i: (0, 0)),
            pl.BlockSpec((4 * H, 1), lambda i: (0, 0)),
        ],
        out_specs=[
            pl.BlockSpec((H, 1), lambda i: (0, 0)),
            pl.BlockSpec((H, B), lambda i: (0, i)),
        ],
        out_shape=[
            jax.ShapeDtypeStruct((H, 1), jnp.float32),
            jax.ShapeDtypeStruct((H, N), jnp.float32),
        ],
        compiler_params=pltpu.CompilerParams(
            dimension_semantics=("arbitrary",)),
    )(scalars, wt_col, bt_col, xT, hT, dT, A_T, bcol)

    return (out_col[:, 0], h_newT.T)


# speedup vs baseline: 6.7835x; 1.0168x over previous
import jax
import jax.numpy as jnp
from jax.experimental import pallas as pl
from jax.experimental.pallas import tpu as pltpu

_H = 64


def kernel(interval, t, co_embeddings, divided, no_embeddings,
           unrelated_embeddings, is_last, hidden_state, W_ih, W_hh, b_ih,
           b_hh, W_t, b_t):
    N, G = co_embeddings.shape
    H = W_hh.shape[1]
    if hidden_state is None:
        hidden_state = jnp.zeros((N, H), co_embeddings.dtype)
    B = 14336
    grid = -(-N // B)

    Z = jnp.zeros((H, H), jnp.float32)
    A_T = jnp.concatenate([
        jnp.concatenate([W_ih[0:H], W_hh[0:H]], axis=1),
        jnp.concatenate([W_ih[H:2 * H], W_hh[H:2 * H]], axis=1),
        jnp.concatenate([W_ih[2 * H:], Z], axis=1),
        jnp.concatenate([Z, W_hh[2 * H:]], axis=1),
    ], axis=0).astype(jnp.bfloat16)

    b_row = jnp.concatenate([
        b_ih[:H] + b_hh[:H], b_ih[H:2 * H] + b_hh[H:2 * H],
        b_ih[2 * H:], b_hh[2 * H:]])
    b_bc = jnp.broadcast_to(b_row[:, None], (4 * H, 128))
    wtbt = jnp.broadcast_to(W_t, (H, 128))
    btbc = jnp.broadcast_to(b_t[:, None], (H, 128))

    interval_s = jnp.asarray(interval, jnp.float32).reshape(1)
    active_s = jnp.logical_not(is_last).astype(jnp.float32).reshape(1)

    xT = co_embeddings.T
    hT = hidden_state.T
    dT = divided.T

    def body(int_ref, act_ref, wt_ref, bt_ref, b_ref, x_ref, h_ref, d_ref,
             a_ref, out_ref, hnew_ref):
        i = pl.program_id(0)

        @pl.when(i == 0)
        def _init():
            out_ref[...] = jnp.full(out_ref.shape, -jnp.inf, jnp.float32)

        h = h_ref[...]
        xh = jnp.concatenate([x_ref[...].astype(jnp.bfloat16),
                              h.astype(jnp.bfloat16)], axis=0)
        g = jax.lax.dot_general(
            a_ref[...], xh, (((1,), (0,)), ((), ())),
            preferred_element_type=jnp.float32) + b_ref[:, 0:1]
        rz = 0.5 * jnp.tanh(0.5 * g[0:2 * H]) + 0.5
        r = rz[0:H]
        z = rz[H:2 * H]
        n = jnp.tanh(g[2 * H:3 * H] + r * g[3 * H:4 * H])
        h_all = n + z * (h - n)

        dmax = jnp.max(d_ref[...], axis=0, keepdims=True)
        col = i * B + jax.lax.broadcasted_iota(jnp.int32, dmax.shape, 1)
        mask = (dmax > 0.0) & (act_ref[0] > 0.0) & (col < N)
        hnew_ref[...] = jnp.where(mask, h_all, 0.0)

        block_max = jnp.max(jnp.where(mask, h_all, -jnp.inf), axis=1,
                            keepdims=True)
        out_ref[...] = jnp.maximum(out_ref[...], block_max)

        @pl.when(i == grid - 1)
        def _finalize():
            inv = 1.0 / jnp.log(int_ref[0] + jnp.exp(1.0))
            tf = jnp.tanh(inv * wt_ref[:, 0:1] + bt_ref[:, 0:1])
            out_ref[...] = out_ref[...] + tf

    out_col, h_newT = pl.pallas_call(
        body,
        grid=(grid,),
        in_specs=[
            pl.BlockSpec(memory_space=pltpu.SMEM),
            pl.BlockSpec(memory_space=pltpu.SMEM),
            pl.BlockSpec((H, 128), lambda i: (0, 0)),
            pl.BlockSpec((H, 128), lambda i: (0, 0)),
            pl.BlockSpec((4 * H, 128), lambda i: (0, 0)),
            pl.BlockSpec((G, B), lambda i: (0, i)),
            pl.BlockSpec((H, B), lambda i: (0, i)),
            pl.BlockSpec((3, B), lambda i: (0, i)),
            pl.BlockSpec((4 * H, G + H), lambda i: (0, 0)),
        ],
        out_specs=[
            pl.BlockSpec((H, 1), lambda i: (0, 0)),
            pl.BlockSpec((H, B), lambda i: (0, i)),
        ],
        out_shape=[
            jax.ShapeDtypeStruct((H, 1), jnp.float32),
            jax.ShapeDtypeStruct((H, N), jnp.float32),
        ],
        compiler_params=pltpu.CompilerParams(
            dimension_semantics=("arbitrary",)),
    )(interval_s, active_s, wtbt, btbc, b_bc, xT, hT, dT, A_T)

    return (out_col[:, 0], h_newT.T)


# speedup vs baseline: 7.9575x; 1.1731x over previous
import jax
import jax.numpy as jnp
from jax.experimental import pallas as pl
from jax.experimental.pallas import tpu as pltpu

_H = 64


def kernel(interval, t, co_embeddings, divided, no_embeddings,
           unrelated_embeddings, is_last, hidden_state, W_ih, W_hh, b_ih,
           b_hh, W_t, b_t):
    N, G = co_embeddings.shape
    H = W_hh.shape[1]
    if hidden_state is None:
        hidden_state = jnp.zeros((N, H), co_embeddings.dtype)
    B = 14336
    grid = -(-N // B)

    interval_s = jnp.asarray(interval, jnp.float32).reshape(1)
    active_s = jnp.logical_not(is_last).astype(jnp.float32).reshape(1)

    xT = co_embeddings.T
    hT = hidden_state.T
    dT = divided.T
    wiT = W_ih.T
    whT = W_hh.T
    bi_row = b_ih[None, :]
    bh_row = b_hh[None, :]
    wt_row = W_t.T
    bt_row = b_t[None, :]

    def body(int_ref, act_ref, wi_ref, wh_ref, bi_ref, bh_ref, wt_ref,
             bt_ref, x_ref, h_ref, d_ref, out_ref, hnew_ref, a_ref, b_ref):
        i = pl.program_id(0)

        @pl.when(i == 0)
        def _init():
            out_ref[...] = jnp.full(out_ref.shape, -jnp.inf, jnp.float32)
            wi = wi_ref[...].astype(jnp.bfloat16)
            wh = wh_ref[...].astype(jnp.bfloat16)
            zb = jnp.zeros((H, H), jnp.bfloat16)
            a_ref[0:H, :] = jnp.concatenate([wi, zb], axis=1)
            a_ref[H:2 * H, 0:2 * H] = wh[:, 0:2 * H]
            a_ref[H:2 * H, 2 * H:3 * H] = zb
            a_ref[H:2 * H, 3 * H:4 * H] = wh[:, 2 * H:3 * H]
            bi = bi_ref[...]
            bh = bh_ref[...]
            stack = jnp.concatenate([bi + bh, bi, bh], axis=0)
            tr = jnp.transpose(stack, (1, 0))
            b_ref[0:2 * H, :] = tr[0:2 * H, 0:1]
            b_ref[2 * H:3 * H, :] = tr[2 * H:3 * H, 1:2]
            b_ref[3 * H:4 * H, :] = tr[2 * H:3 * H, 2:3]

        h = h_ref[...]
        xh = jnp.concatenate([x_ref[...].astype(jnp.bfloat16),
                              h.astype(jnp.bfloat16)], axis=0)
        g = jax.lax.dot_general(
            a_ref[...], xh, (((0,), (0,)), ((), ())),
            preferred_element_type=jnp.float32) + b_ref[...]
        rz = 0.5 * jnp.tanh(0.5 * g[0:2 * H]) + 0.5
        r = rz[0:H]
        z = rz[H:2 * H]
        n = jnp.tanh(g[2 * H:3 * H] + r * g[3 * H:4 * H])
        h_all = n + z * (h - n)

        dmax = jnp.max(d_ref[...], axis=0, keepdims=True)
        col = i * B + jax.lax.broadcasted_iota(jnp.int32, dmax.shape, 1)
        mask = (dmax > 0.0) & (act_ref[0] > 0.0) & (col < N)
        hnew_ref[...] = jnp.where(mask, h_all, 0.0)

        block_max = jnp.max(jnp.where(mask, h_all, -jnp.inf), axis=1,
                            keepdims=True)
        out_ref[...] = jnp.maximum(out_ref[...], block_max)

        @pl.when(i == grid - 1)
        def _finalize():
            inv = 1.0 / jnp.log(int_ref[0] + jnp.exp(1.0))
            wtbt = jnp.transpose(
                jnp.concatenate([wt_row_scale(wt_ref, inv), bt_ref[...]],
                                axis=0), (1, 0))
            tf = jnp.tanh(wtbt[:, 0:1] + wtbt[:, 1:2])
            out_ref[...] = out_ref[...] + tf

    def wt_row_scale(wt_ref, inv):
        return inv * wt_ref[...]

    out_col, h_newT = pl.pallas_call(
        body,
        grid=(grid,),
        in_specs=[
            pl.BlockSpec(memory_space=pltpu.SMEM),
            pl.BlockSpec(memory_space=pltpu.SMEM),
            pl.BlockSpec((G, 3 * H), lambda i: (0, 0)),
            pl.BlockSpec((H, 3 * H), lambda i: (0, 0)),
            pl.BlockSpec((1, 3 * H), lambda i: (0, 0)),
            pl.BlockSpec((1, 3 * H), lambda i: (0, 0)),
            pl.BlockSpec((1, H), lambda i: (0, 0)),
            pl.BlockSpec((1, H), lambda i: (0, 0)),
            pl.BlockSpec((G, B), lambda i: (0, i)),
            pl.BlockSpec((H, B), lambda i: (0, i)),
            pl.BlockSpec((3, B), lambda i: (0, i)),
        ],
        out_specs=[
            pl.BlockSpec((H, 1), lambda i: (0, 0)),
            pl.BlockSpec((H, B), lambda i: (0, i)),
        ],
        out_shape=[
            jax.ShapeDtypeStruct((H, 1), jnp.float32),
            jax.ShapeDtypeStruct((H, N), jnp.float32),
        ],
        scratch_shapes=[
            pltpu.VMEM((2 * H, 4 * H), jnp.bfloat16),
            pltpu.VMEM((4 * H, 1), jnp.float32),
        ],
        compiler_params=pltpu.CompilerParams(
            dimension_semantics=("arbitrary",)),
    )(interval_s, active_s, wiT, whT, bi_row, bh_row, wt_row, bt_row,
      xT, hT, dT)

    return (out_col[:, 0], h_newT.T)
